# Initial kernel scaffold; baseline (speedup 1.0000x reference)
import functools

import jax
import jax.numpy as jnp
from jax import lax
from jax.experimental import pallas as pl
from jax.experimental.pallas import tpu as pltpu
from jax.experimental.pallas import tpu_sc as plsc

NC = 2
NS = 16
CHUNK = 128


def _make_sc_agg(n_table, feat, e_pad, n_acc, edge_split):
  per_tile = e_pad // (NC * NS) if edge_split else e_pad // NS
  n_chunks = per_tile // CHUNK
  rows_per_tile = n_acc // NS
  nz = rows_per_tile // 128
  mesh = plsc.VectorSubcoreMesh(core_axis_name="c", subcore_axis_name="s")

  @functools.partial(
      pl.kernel,
      out_type=jax.ShapeDtypeStruct((NC, n_acc, feat), jnp.float32),
      mesh=mesh,
      scratch_types=[
          pltpu.VMEM((CHUNK,), jnp.int32),
          pltpu.VMEM((CHUNK,), jnp.int32),
          pltpu.VMEM((CHUNK, feat), jnp.float32),
          pltpu.VMEM((128, feat), jnp.float32),
          pltpu.VMEM_SHARED((n_acc, feat), jnp.float32),
          pltpu.SemaphoreType.DMA,
      ],
  )
  def k(table_hbm, src_hbm, dst_hbm, zeros_hbm, out_hbm,
        src_v, dst_v, rows_v, zbuf, acc_sh, sem):
    c = lax.axis_index("c")
    s = lax.axis_index("s")
    if edge_split:
      base = (c * NS + s) * per_tile
    else:
      base = s * per_tile
    row0 = s * rows_per_tile

    pltpu.sync_copy(zeros_hbm, zbuf)
    for z in range(nz):
      pltpu.sync_copy(zbuf, acc_sh.at[pl.ds(row0 + z * 128, 128)])
    plsc.subcore_barrier()

    def body(j, carry):
      off = base + j * CHUNK
      pltpu.sync_copy(src_hbm.at[c, pl.ds(off, CHUNK)], src_v)
      pltpu.sync_copy(dst_hbm.at[pl.ds(off, CHUNK)], dst_v)
      pltpu.async_copy(table_hbm.at[src_v], rows_v, sem).wait()
      pltpu.sync_copy(rows_v, acc_sh.at[dst_v], add=True)
      return carry

    lax.fori_loop(0, n_chunks, body, 0)
    plsc.subcore_barrier()

    for z in range(nz):
      r = row0 + z * 128
      pltpu.sync_copy(acc_sh.at[pl.ds(r, 128)], rows_v)
      pltpu.sync_copy(rows_v, out_hbm.at[c, pl.ds(r, 128)])

  return k


def _tc_layer1(parts, x_pad, wl, wr, b, n_acc, in_dim, h_dim):
  BN = 512
  feat = parts.shape[2]
  hh = h_dim // 2

  def body(parts_ref, x_ref, wl_ref, wr_ref, b_ref, h1_ref, invd_ref):
    p = parts_ref[0] + parts_ref[1]
    invd = 1.0 / jnp.maximum(p[:, in_dim:in_dim + 1], 1.0)
    mean = p[:, :in_dim] * invd
    h = jnp.dot(mean, wl_ref[...], preferred_element_type=jnp.float32)
    h += jnp.dot(x_ref[...], wr_ref[...], preferred_element_type=jnp.float32)
    h = jnp.maximum(h + b_ref[...], 0.0)
    h1_ref[0] = h[:, :hh]
    h1_ref[1] = h[:, hh:]
    invd_ref[...] = invd[:, 0]

  return pl.pallas_call(
      body,
      grid=(n_acc // BN,),
      in_specs=[
          pl.BlockSpec((NC, BN, feat), lambda i: (0, i, 0)),
          pl.BlockSpec((BN, in_dim), lambda i: (i, 0)),
          pl.BlockSpec((in_dim, h_dim), lambda i: (0, 0)),
          pl.BlockSpec((in_dim, h_dim), lambda i: (0, 0)),
          pl.BlockSpec((h_dim,), lambda i: (0,)),
      ],
      out_specs=[
          pl.BlockSpec((NC, BN, hh), lambda i: (0, i, 0)),
          pl.BlockSpec((BN,), lambda i: (i,)),
      ],
      out_shape=[
          jax.ShapeDtypeStruct((NC, n_acc, hh), jnp.float32),
          jax.ShapeDtypeStruct((n_acc,), jnp.float32),
      ],
  )(parts, x_pad, wl, wr, b)


def _tc_layer2(parts2, h1, invd, wl2, wr2, b2, wc_pad, bc_pad, n_acc, h_dim):
  BN = 512
  hh = h_dim // 2
  oc = wc_pad.shape[1]

  def body(p2_ref, h1_ref, invd_ref, wl_ref, wr_ref, b_ref, wc_ref, bc_ref,
           out_ref):
    agg = jnp.concatenate([p2_ref[0], p2_ref[1]], axis=1)
    mean = agg * invd_ref[...][:, None]
    hp = jnp.concatenate([h1_ref[0], h1_ref[1]], axis=1)
    h = jnp.dot(mean, wl_ref[...], preferred_element_type=jnp.float32)
    h += jnp.dot(hp, wr_ref[...], preferred_element_type=jnp.float32)
    h = jnp.maximum(h + b_ref[...], 0.0)
    out_ref[...] = (
        jnp.dot(h, wc_ref[...], preferred_element_type=jnp.float32)
        + bc_ref[...])

  return pl.pallas_call(
      body,
      grid=(n_acc // BN,),
      in_specs=[
          pl.BlockSpec((NC, BN, hh), lambda i: (0, i, 0)),
          pl.BlockSpec((NC, BN, hh), lambda i: (0, i, 0)),
          pl.BlockSpec((BN,), lambda i: (i,)),
          pl.BlockSpec((h_dim, h_dim), lambda i: (0, 0)),
          pl.BlockSpec((h_dim, h_dim), lambda i: (0, 0)),
          pl.BlockSpec((h_dim,), lambda i: (0,)),
          pl.BlockSpec((h_dim, oc), lambda i: (0, 0)),
          pl.BlockSpec((oc,), lambda i: (0,)),
      ],
      out_specs=pl.BlockSpec((BN, oc), lambda i: (i, 0)),
      out_shape=jax.ShapeDtypeStruct((n_acc, oc), jnp.float32),
  )(parts2, h1, invd, wl2, wr2, b2, wc_pad, bc_pad)


def kernel(x, edge_index, W_l1, W_r1, b1, W_l2, W_r2, b2, Wc, bc):
  n, in_dim = x.shape
  e = edge_index.shape[1]
  h_dim = W_l1.shape[1]
  out_dim = Wc.shape[1]
  hh = h_dim // 2

  e_pad = -(-e // (NC * NS * CHUNK)) * (NC * NS * CHUNK)
  n_acc = -(-(n + 1) // (NS * 128)) * (NS * 128)

  src = edge_index[0]
  dst = edge_index[1]
  pad = e_pad - e
  src_p = jnp.concatenate([src, jnp.zeros((pad,), jnp.int32)])
  dst_p = jnp.concatenate([dst, jnp.full((pad,), n, jnp.int32)])

  feat1 = -(-(in_dim + 1) // 16) * 16
  table1 = jnp.concatenate(
      [x, jnp.ones((n, 1), jnp.float32),
       jnp.zeros((n, feat1 - in_dim - 1), jnp.float32)], axis=1)
  src2_1 = jnp.stack([src_p, src_p])
  zeros1 = jnp.zeros((128, feat1), jnp.float32)

  agg1 = _make_sc_agg(n, feat1, e_pad, n_acc, edge_split=True)
  parts1 = agg1(table1, src2_1, dst_p, zeros1)

  x_pad = jnp.concatenate(
      [x, jnp.zeros((n_acc - n, in_dim), jnp.float32)], axis=0)
  h1, invd = _tc_layer1(parts1, x_pad, W_l1, W_r1, b1, n_acc, in_dim, h_dim)

  table2 = h1.reshape(NC * n_acc, hh)
  src2_2 = jnp.stack([src_p, src_p + n_acc])
  zeros2 = jnp.zeros((128, hh), jnp.float32)

  agg2 = _make_sc_agg(NC * n_acc, hh, e_pad, n_acc, edge_split=False)
  parts2 = agg2(table2, src2_2, dst_p, zeros2)

  oc = 128
  wc_pad = jnp.zeros((h_dim, oc), jnp.float32).at[:, :out_dim].set(Wc)
  bc_pad = jnp.zeros((oc,), jnp.float32).at[:out_dim].set(bc)
  out = _tc_layer2(parts2, h1, invd, W_l2, W_r2, b2, wc_pad, bc_pad,
                   n_acc, h_dim)
  return out[:n, :out_dim]


# speedup vs baseline: 3.9236x; 3.9236x over previous
import functools

import jax
import jax.numpy as jnp
from jax import lax
from jax.experimental import pallas as pl
from jax.experimental.pallas import tpu as pltpu
from jax.experimental.pallas import tpu_sc as plsc

NC = 2
NS = 16
CHUNK = 128
DCOL = 2


def _make_sc_agg(feat, e_pad, n_acc, edge_split, want_deg):
  per_tile = e_pad // (NC * NS) if edge_split else e_pad // NS
  n_chunks = per_tile // CHUNK
  rows_per_tile = n_acc // NS
  nz = rows_per_tile // 128
  mesh = plsc.VectorSubcoreMesh(core_axis_name="c", subcore_axis_name="s")

  out_type = [jax.ShapeDtypeStruct((NC, n_acc, feat), jnp.float32)]
  scratch = [
      pltpu.VMEM((CHUNK,), jnp.int32),
      pltpu.VMEM((CHUNK,), jnp.int32),
      pltpu.VMEM((CHUNK, feat), jnp.float32),
      pltpu.VMEM_SHARED((n_acc, feat), jnp.float32),
      pltpu.SemaphoreType.DMA,
  ]
  if want_deg:
    out_type.append(jax.ShapeDtypeStruct((NC, NS, n_acc), jnp.float32))
    scratch.append(pltpu.VMEM((DCOL * n_acc,), jnp.float32))

  @functools.partial(
      pl.kernel, out_type=tuple(out_type), mesh=mesh, scratch_types=scratch,
      compiler_params=pltpu.CompilerParams(needs_layout_passes=False))
  def k(*refs):
    if want_deg:
      (table_hbm, src_hbm, dst_hbm, zeros_hbm, zdeg_hbm,
       out_hbm, odeg_hbm, src_v, dst_v, rows_v, acc_sh, sem,
       deg_v) = refs
    else:
      (table_hbm, src_hbm, dst_hbm, zeros_hbm,
       out_hbm, src_v, dst_v, rows_v, acc_sh, sem) = refs

    c = lax.axis_index("c")
    s = lax.axis_index("s")
    if edge_split:
      base = (c * NS + s) * per_tile
    else:
      base = s * per_tile
    row0 = s * rows_per_tile

    for z in range(nz):
      pltpu.sync_copy(zeros_hbm, acc_sh.at[pl.ds(row0 + z * 128, 128)])
    if want_deg:
      pltpu.sync_copy(zdeg_hbm, deg_v)
      lane = lax.iota(jnp.int32, 16)
      colbase = (lane & (DCOL - 1)) * n_acc
      ones16 = jnp.full((16,), 1.0, jnp.float32)
      gmasks = [(lane // DCOL) == g for g in range(16 // DCOL)]
    plsc.subcore_barrier()

    def body(j, carry):
      off = base + j * CHUNK
      pltpu.sync_copy(src_hbm.at[c, pl.ds(off, CHUNK)], src_v)
      pltpu.sync_copy(dst_hbm.at[pl.ds(off, CHUNK)], dst_v)
      pltpu.async_copy(table_hbm.at[src_v], rows_v, sem).wait()
      pltpu.sync_copy(rows_v, acc_sh.at[dst_v], add=True)
      if want_deg:
        for q in range(CHUNK // 16):
          dstv = dst_v[pl.ds(q * 16, 16)]
          idxv = dstv + colbase
          for m in gmasks:
            plsc.addupdate_scatter(deg_v, [idxv], ones16, mask=m)
      return carry

    lax.fori_loop(0, n_chunks, body, 0)
    plsc.subcore_barrier()

    for z in range(nz):
      r = row0 + z * 128
      pltpu.sync_copy(acc_sh.at[pl.ds(r, 128)], out_hbm.at[c, pl.ds(r, 128)])

    if want_deg:
      def red(i, carry):
        v = deg_v[pl.ds(i * 16, 16)]
        for d in range(1, DCOL):
          v += deg_v[pl.ds(d * n_acc + i * 16, 16)]
        deg_v[pl.ds(i * 16, 16)] = v
        return carry
      lax.fori_loop(0, n_acc // 16, red, 0)
      pltpu.sync_copy(deg_v.at[pl.ds(0, n_acc)], odeg_hbm.at[c, s])

  return k


def _tc_layer1(parts, degp, x_pad, wl, wr, b, n_acc, in_dim, h_dim):
  BN = 512
  feat = parts.shape[2]
  hh = h_dim // 2

  def body(parts_ref, degp_ref, x_ref, wl_ref, wr_ref, b_ref,
           h1_ref, invd_ref):
    deg = jnp.sum(degp_ref[...], axis=(0, 1))
    invd = 1.0 / jnp.maximum(deg, 1.0)
    p = parts_ref[0] + parts_ref[1]
    mean = p * invd[:, None]
    h = jnp.dot(mean, wl_ref[...], preferred_element_type=jnp.float32)
    h += jnp.dot(x_ref[...], wr_ref[...], preferred_element_type=jnp.float32)
    h = jnp.maximum(h + b_ref[...], 0.0)
    h1_ref[0] = h[:, :hh]
    h1_ref[1] = h[:, hh:]
    invd_ref[...] = invd

  return pl.pallas_call(
      body,
      grid=(n_acc // BN,),
      in_specs=[
          pl.BlockSpec((NC, BN, feat), lambda i: (0, i, 0)),
          pl.BlockSpec((NC, NS, BN), lambda i: (0, 0, i)),
          pl.BlockSpec((BN, in_dim), lambda i: (i, 0)),
          pl.BlockSpec((in_dim, h_dim), lambda i: (0, 0)),
          pl.BlockSpec((in_dim, h_dim), lambda i: (0, 0)),
          pl.BlockSpec((h_dim,), lambda i: (0,)),
      ],
      out_specs=[
          pl.BlockSpec((NC, BN, hh), lambda i: (0, i, 0)),
          pl.BlockSpec((BN,), lambda i: (i,)),
      ],
      out_shape=[
          jax.ShapeDtypeStruct((NC, n_acc, hh), jnp.float32),
          jax.ShapeDtypeStruct((n_acc,), jnp.float32),
      ],
  )(parts, degp, x_pad, wl, wr, b)


def _tc_layer2(parts2, h1, invd, wl2, wr2, b2, wc_pad, bc_pad, n_acc, h_dim):
  BN = 512
  hh = h_dim // 2
  oc = wc_pad.shape[1]

  def body(p2_ref, h1_ref, invd_ref, wl_ref, wr_ref, b_ref, wc_ref, bc_ref,
           out_ref):
    agg = jnp.concatenate([p2_ref[0], p2_ref[1]], axis=1)
    mean = agg * invd_ref[...][:, None]
    hp = jnp.concatenate([h1_ref[0], h1_ref[1]], axis=1)
    h = jnp.dot(mean, wl_ref[...], preferred_element_type=jnp.float32)
    h += jnp.dot(hp, wr_ref[...], preferred_element_type=jnp.float32)
    h = jnp.maximum(h + b_ref[...], 0.0)
    out_ref[...] = (
        jnp.dot(h, wc_ref[...], preferred_element_type=jnp.float32)
        + bc_ref[...])

  return pl.pallas_call(
      body,
      grid=(n_acc // BN,),
      in_specs=[
          pl.BlockSpec((NC, BN, hh), lambda i: (0, i, 0)),
          pl.BlockSpec((NC, BN, hh), lambda i: (0, i, 0)),
          pl.BlockSpec((BN,), lambda i: (i,)),
          pl.BlockSpec((h_dim, h_dim), lambda i: (0, 0)),
          pl.BlockSpec((h_dim, h_dim), lambda i: (0, 0)),
          pl.BlockSpec((h_dim,), lambda i: (0,)),
          pl.BlockSpec((h_dim, oc), lambda i: (0, 0)),
          pl.BlockSpec((oc,), lambda i: (0,)),
      ],
      out_specs=pl.BlockSpec((BN, oc), lambda i: (i, 0)),
      out_shape=jax.ShapeDtypeStruct((n_acc, oc), jnp.float32),
  )(parts2, h1, invd, wl2, wr2, b2, wc_pad, bc_pad)


def kernel(x, edge_index, W_l1, W_r1, b1, W_l2, W_r2, b2, Wc, bc):
  n, in_dim = x.shape
  e = edge_index.shape[1]
  h_dim = W_l1.shape[1]
  out_dim = Wc.shape[1]
  hh = h_dim // 2

  e_pad = -(-e // (NC * NS * CHUNK)) * (NC * NS * CHUNK)
  n_acc = -(-(n + 1) // (NS * 128)) * (NS * 128)

  src = edge_index[0]
  dst = edge_index[1]
  pad = e_pad - e
  src_p = jnp.concatenate([src, jnp.zeros((pad,), jnp.int32)])
  dst_p = jnp.concatenate([dst, jnp.full((pad,), n, jnp.int32)])

  src2_1 = jnp.stack([src_p, src_p])
  zeros1 = jnp.zeros((128, in_dim), jnp.float32)
  zdeg = jnp.zeros((DCOL * n_acc,), jnp.float32)

  agg1 = _make_sc_agg(in_dim, e_pad, n_acc, edge_split=True, want_deg=True)
  parts1, degp = agg1(x, src2_1, dst_p, zeros1, zdeg)

  x_pad = jnp.concatenate(
      [x, jnp.zeros((n_acc - n, in_dim), jnp.float32)], axis=0)
  h1, invd = _tc_layer1(parts1, degp, x_pad, W_l1, W_r1, b1,
                        n_acc, in_dim, h_dim)

  table2 = h1.reshape(NC * n_acc, hh)
  src2_2 = jnp.stack([src_p, src_p + n_acc])
  zeros2 = jnp.zeros((128, hh), jnp.float32)

  agg2 = _make_sc_agg(hh, e_pad, n_acc, edge_split=False, want_deg=False)
  (parts2,) = agg2(table2, src2_2, dst_p, zeros2)

  oc = 128
  wc_pad = jnp.zeros((h_dim, oc), jnp.float32).at[:, :out_dim].set(Wc)
  bc_pad = jnp.zeros((oc,), jnp.float32).at[:out_dim].set(bc)
  out = _tc_layer2(parts2, h1, invd, W_l2, W_r2, b2, wc_pad, bc_pad,
                   n_acc, h_dim)
  return out[:n, :out_dim]
